# Initial kernel scaffold; baseline (speedup 1.0000x reference)
import jax
import jax.numpy as jnp
from jax.experimental import pallas as pl


def kernel(det_boxes, det_scores, translations):
    raise NotImplementedError("write your pallas kernel here")


# speedup vs baseline: 60.0540x; 60.0540x over previous
import jax
import jax.numpy as jnp
from jax.experimental import pallas as pl
from jax.experimental.pallas import tpu as pltpu

PI_C = 3.141592653
NAG, NPER_C = 4, 256
NB = NAG * NPER_C
THR = 0.1
TILE = 256
CHK = 128

_INTERPRET = False

_SGN = ((0.5, 0.5), (-0.5, 0.5), (-0.5, -0.5), (0.5, -0.5))


def _limit_period(v):
    return v - jnp.floor(v / (2.0 * PI_C) + 0.5) * (2.0 * PI_C)


def _agent_shift(t_ref, sel_mask):
    tx = t_ref[:, 0:1]
    ty = t_ref[:, 1:2]
    a4 = jax.lax.broadcasted_iota(jnp.int32, (4, 1), 0)
    m0 = a4 == 0
    sx = (jnp.sum(jnp.where(sel_mask, tx, 0.0), keepdims=True)
          - jnp.sum(jnp.where(m0, tx, 0.0), keepdims=True))
    sy = (jnp.sum(jnp.where(sel_mask, ty, 0.0), keepdims=True)
          - jnp.sum(jnp.where(m0, ty, 0.0), keepdims=True))
    return sx.reshape(1, 1), sy.reshape(1, 1)


def _clip_contrib(px, py, qx, qy, ux, uy, acc):
    dx, dy = qx - px, qy - py
    t_lo = None
    t_hi = None
    infea = None
    for k in range(4):
        k2 = (k + 1) % 4
        ex = ux[k2] - ux[k]
        ey = uy[k2] - uy[k]
        beta = ex * dy - ey * dx
        alpha = ex * (py - uy[k]) - ey * (px - ux[k])
        r = -alpha / jnp.where(beta == 0.0, 1.0, beta)
        lo_k = jnp.where(beta > 0.0, r, -1e9)
        hi_k = jnp.where(beta < 0.0, r, 1e9)
        bad_k = (beta == 0.0) & (alpha < 0.0)
        if t_lo is None:
            t_lo, t_hi, infea = lo_k, hi_k, bad_k
        else:
            t_lo = jnp.maximum(t_lo, lo_k)
            t_hi = jnp.minimum(t_hi, hi_k)
            infea = infea | bad_k
    t_lo = jnp.maximum(t_lo, 0.0)
    t_hi = jnp.minimum(t_hi, 1.0)
    t_lo = jnp.where(infea, 1e9, t_lo)
    return acc + jnp.where(t_hi > t_lo, (t_hi - t_lo) * (px * dy - py * dx), 0.0)


def _corners(x, y, dx, dy, h):
    c, s = jnp.cos(h), jnp.sin(h)
    cxs, cys = [], []
    for sx, sy in _SGN:
        lx = sx * dx
        ly = sy * dy
        cxs.append(c * lx - s * ly + x)
        cys.append(s * lx + c * ly + y)
    return cxs, cys


def _adj_kernel(br_ref, bcT_ref, t_ref, adj_ref):
    i = pl.program_id(0)
    j = pl.program_id(1)
    a4 = jax.lax.broadcasted_iota(jnp.int32, (4, 1), 0)
    shxr, shyr = _agent_shift(t_ref, a4 == i)
    shxc, shyc = _agent_shift(t_ref, a4 == j)

    br = br_ref[:, :]
    xr = br[:, 0:1] + shxr
    yr = br[:, 1:2] + shyr
    zr, dxr, dyr, dzr, hr = (br[:, 2:3], br[:, 3:4], br[:, 4:5], br[:, 5:6],
                             br[:, 6:7])
    bc = bcT_ref[:, :]
    xc = bc[0:1, :] + shxc
    yc = bc[1:2, :] + shyc
    zc, dxc, dyc, dzc, hc = (bc[2:3, :], bc[3:4, :], bc[4:5, :], bc[5:6, :],
                             bc[6:7, :])

    arx, ary = _corners(xr, yr, dxr, dyr, hr)
    bcx, bcy = _corners(xc, yc, dxc, dyc, hc)

    mx = 0.5 * (xr + xc)
    my = 0.5 * (yr + yc)
    acx = [a - mx for a in arx]
    acy = [a - my for a in ary]
    ccx = [b - mx for b in bcx]
    ccy = [b - my for b in bcy]

    acc1 = jnp.zeros_like(mx)
    acc2 = jnp.zeros_like(mx)
    for e in range(4):
        e2 = (e + 1) % 4
        acc1 = _clip_contrib(acx[e], acy[e], acx[e2], acy[e2],
                             tuple(ccx), tuple(ccy), acc1)
        acc2 = _clip_contrib(ccx[e], ccy[e], ccx[e2], ccy[e2],
                             tuple(acx), tuple(acy), acc2)
    inter = 0.5 * (acc1 + acc2)

    ih = jnp.maximum(jnp.minimum(zr + dzr * 0.5, zc + dzc * 0.5)
                     - jnp.maximum(zr - dzr * 0.5, zc - dzc * 0.5), 0.0)
    iv = inter * ih
    va = dxr * dyr * dzr
    vb = dxc * dyc * dzc
    iou = iv / jnp.maximum(va + vb - iv, 1e-6)
    adj_ref[:, :] = (iou > THR).astype(jnp.float32)


def _fuse_kernel(adj_ref, b_ref, bT_ref, sc_ref, sr_ref, t_ref, out_ref):
    f32 = jnp.float32
    lane_n = jax.lax.broadcasted_iota(jnp.int32, (1, NB), 1)
    sub_n = jax.lax.broadcasted_iota(jnp.int32, (NB, 1), 0)

    covered = jnp.zeros((1, NB), f32)
    lead_chunks = []
    lane_c = jax.lax.broadcasted_iota(jnp.int32, (1, CHK), 1)
    sub_c = jax.lax.broadcasted_iota(jnp.int32, (CHK, 1), 0)
    sub_8 = jax.lax.broadcasted_iota(jnp.int32, (8, 1), 0)
    for c0 in range(0, NB, CHK):
        def body(k, st):
            covc, leadr, leadcol = st
            oh = lane_c == k
            ohT = sub_c == k
            covk = jnp.sum(jnp.where(oh, covc, 0.0), axis=1, keepdims=True)
            isl = covk == 0.0
            base = pl.multiple_of(((c0 + k) >> 3) << 3, 8)
            blk = adj_ref[pl.ds(base, 8), pl.ds(c0, CHK)]
            row = jnp.max(jnp.where(sub_8 == ((c0 + k) & 7), blk, 0.0),
                          axis=0, keepdims=True)
            covc = jnp.where(isl & (row > 0.0), 1.0, covc)
            leadr = jnp.where(isl & oh, 1.0, leadr)
            leadcol = jnp.where(isl & ohT, 1.0, leadcol)
            return covc, leadr, leadcol

        covc0 = jax.lax.slice(covered, (0, c0), (1, c0 + CHK))
        _, leadr, leadcol = jax.lax.fori_loop(
            0, CHK, body, (covc0, jnp.zeros((1, CHK), f32),
                           jnp.zeros((CHK, 1), f32)))
        lead_chunks.append(leadr)
        rows = adj_ref[pl.ds(c0, CHK), :]
        covered = jnp.maximum(
            covered,
            jnp.max(jnp.where(leadcol > 0.0, rows, 0.0), axis=0,
                    keepdims=True))
    leader_row = jnp.concatenate(lead_chunks, axis=1)

    idn = sub_n == lane_n
    leaderT = jnp.sum(jnp.where(idn, leader_row, 0.0), axis=1, keepdims=True)
    cumT = jnp.sum(jnp.where(lane_n <= sub_n, leader_row, 0.0), axis=1,
                   keepdims=True)
    valT = leaderT * cumT
    adjv = adj_ref[:, :]
    segf_row = (jnp.max(jnp.where(adjv > 0.0, valT, 0.0), axis=0,
                        keepdims=True) - 1.0)
    segT = jnp.sum(jnp.where(idn, segf_row, 0.0), axis=1, keepdims=True)
    same = segT == segf_row
    samef = same.astype(f32)

    s_row = sr_ref[:, :]
    s_col = sc_ref[:, :]

    smax = jnp.max(jnp.where(same, s_row, 0.0), axis=1, keepdims=True)
    sel = same & (s_row == smax)
    lane_f = lane_n.astype(f32)
    amin = jnp.min(jnp.where(sel, lane_f, float(NB)), axis=1, keepdims=True)
    refm = (sel & (lane_f == amin)).astype(f32)
    dirs_row = bT_ref[6:7, :]
    ref_dir = jnp.sum(refm * dirs_row, axis=1, keepdims=True)

    dirsT = b_ref[:, 6:7]
    dd = jnp.abs(dirsT - ref_dir)
    dd = jnp.where(dd > PI_C, 2.0 * PI_C - dd, dd)
    mgt = (dd > PI_C / 2.0).astype(f32)

    x1 = jnp.concatenate([s_col * mgt, s_col * (1.0 - mgt), s_col], axis=1)
    s1 = jnp.dot(samef, x1, preferred_element_type=f32)
    sg, sle, ssum = s1[:, 0:1], s1[:, 1:2], s1[:, 2:3]
    addf = jnp.where(sg <= sle, mgt, 1.0 - mgt)
    dirs2 = _limit_period(dirsT + addf * PI_C)
    snorm = s_col / jnp.where(ssum > 0.0, ssum, 1.0)

    gt = (s_row > s_col) | ((s_row == s_col) & (lane_n < sub_n))
    rank = jnp.sum((same & gt).astype(f32), axis=1, keepdims=True)
    term = jnp.exp((rank + 1.0) * jnp.log(s_col))

    mm = (sub_n.astype(f32) == segf_row).astype(f32)

    a4 = jax.lax.broadcasted_iota(jnp.int32, (4, 1), 0)
    ag = sub_n // NPER_C
    shx = jnp.zeros((NB, 1), f32)
    shy = jnp.zeros((NB, 1), f32)
    for a in range(4):
        sx_a, sy_a = _agent_shift(t_ref, a4 == a)
        shx = jnp.where(ag == a, sx_a, shx)
        shy = jnp.where(ag == a, sy_a, shy)
    bx = b_ref[:, 0:1] + shx
    by = b_ref[:, 1:2] + shy

    x2 = jnp.concatenate(
        [bx * snorm, by * snorm, b_ref[:, 2:3] * snorm, b_ref[:, 3:4] * snorm,
         b_ref[:, 4:5] * snorm, b_ref[:, 5:6] * snorm,
         jnp.sin(dirs2) * snorm, jnp.cos(dirs2) * snorm, term], axis=1)
    s2 = jnp.dot(mm, x2, preferred_element_type=f32)
    theta = jnp.arctan2(s2[:, 6:7], s2[:, 7:8])
    sf = jnp.minimum(s2[:, 8:9], 1.0)
    out_ref[:, :] = jnp.concatenate([s2[:, 0:6], theta, sf], axis=1)


def kernel(det_boxes, det_scores, translations):
    f32 = jnp.float32
    boxes = det_boxes.astype(f32).reshape(NB, 7)
    bT = boxes.T
    s = det_scores.astype(f32).reshape(NB)
    s_col = s.reshape(NB, 1)
    s_row = s.reshape(1, NB)
    t = translations.astype(f32)

    g = NB // TILE
    adj = pl.pallas_call(
        _adj_kernel,
        grid=(g, g),
        in_specs=[
            pl.BlockSpec((TILE, 7), lambda i, j: (i, 0)),
            pl.BlockSpec((7, TILE), lambda i, j: (0, j)),
            pl.BlockSpec((4, 3), lambda i, j: (0, 0)),
        ],
        out_specs=pl.BlockSpec((TILE, TILE), lambda i, j: (i, j)),
        out_shape=jax.ShapeDtypeStruct((NB, NB), f32),
        compiler_params=pltpu.CompilerParams(
            dimension_semantics=("parallel", "arbitrary")),
        interpret=_INTERPRET,
    )(boxes, bT, t)

    out = pl.pallas_call(
        _fuse_kernel,
        out_shape=jax.ShapeDtypeStruct((NB, 8), f32),
        interpret=_INTERPRET,
    )(adj, boxes, bT, s_col, s_row, t)
    return out


# speedup vs baseline: 62.3466x; 1.0382x over previous
import jax
import jax.numpy as jnp
from jax.experimental import pallas as pl
from jax.experimental.pallas import tpu as pltpu

PI_C = 3.141592653
NAG, NPER_C = 4, 256
NB = NAG * NPER_C
THR = 0.1
TILE = 256
CHK = 128

_INTERPRET = False

_SGN = ((0.5, 0.5), (-0.5, 0.5), (-0.5, -0.5), (0.5, -0.5))


def _limit_period(v):
    return v - jnp.floor(v / (2.0 * PI_C) + 0.5) * (2.0 * PI_C)


def _agent_shift(t_ref, sel_mask):
    tx = t_ref[:, 0:1]
    ty = t_ref[:, 1:2]
    a4 = jax.lax.broadcasted_iota(jnp.int32, (4, 1), 0)
    m0 = a4 == 0
    sx = (jnp.sum(jnp.where(sel_mask, tx, 0.0), keepdims=True)
          - jnp.sum(jnp.where(m0, tx, 0.0), keepdims=True))
    sy = (jnp.sum(jnp.where(sel_mask, ty, 0.0), keepdims=True)
          - jnp.sum(jnp.where(m0, ty, 0.0), keepdims=True))
    return sx.reshape(1, 1), sy.reshape(1, 1)


def _clip_contrib(px, py, qx, qy, ux, uy, acc):
    dx, dy = qx - px, qy - py
    t_lo = None
    t_hi = None
    infea = None
    for k in range(4):
        k2 = (k + 1) % 4
        ex = ux[k2] - ux[k]
        ey = uy[k2] - uy[k]
        beta = ex * dy - ey * dx
        alpha = ex * (py - uy[k]) - ey * (px - ux[k])
        r = -alpha / jnp.where(beta == 0.0, 1.0, beta)
        lo_k = jnp.where(beta > 0.0, r, -1e9)
        hi_k = jnp.where(beta < 0.0, r, 1e9)
        bad_k = (beta == 0.0) & (alpha < 0.0)
        if t_lo is None:
            t_lo, t_hi, infea = lo_k, hi_k, bad_k
        else:
            t_lo = jnp.maximum(t_lo, lo_k)
            t_hi = jnp.minimum(t_hi, hi_k)
            infea = infea | bad_k
    t_lo = jnp.maximum(t_lo, 0.0)
    t_hi = jnp.minimum(t_hi, 1.0)
    t_lo = jnp.where(infea, 1e9, t_lo)
    return acc + jnp.where(t_hi > t_lo, (t_hi - t_lo) * (px * dy - py * dx), 0.0)


def _corners(x, y, dx, dy, h):
    c, s = jnp.cos(h), jnp.sin(h)
    cxs, cys = [], []
    for sx, sy in _SGN:
        lx = sx * dx
        ly = sy * dy
        cxs.append(c * lx - s * ly + x)
        cys.append(s * lx + c * ly + y)
    return cxs, cys


def _adj_kernel(br_ref, bcT_ref, t_ref, adj_ref):
    i = pl.program_id(0)
    j = pl.program_id(1)
    a4 = jax.lax.broadcasted_iota(jnp.int32, (4, 1), 0)
    shxr, shyr = _agent_shift(t_ref, a4 == i)
    shxc, shyc = _agent_shift(t_ref, a4 == j)

    br = br_ref[:, :]
    xr = br[:, 0:1] + shxr
    yr = br[:, 1:2] + shyr
    zr, dxr, dyr, dzr, hr = (br[:, 2:3], br[:, 3:4], br[:, 4:5], br[:, 5:6],
                             br[:, 6:7])
    bc = bcT_ref[:, :]
    xc = bc[0:1, :] + shxc
    yc = bc[1:2, :] + shyc
    zc, dxc, dyc, dzc, hc = (bc[2:3, :], bc[3:4, :], bc[4:5, :], bc[5:6, :],
                             bc[6:7, :])

    arx, ary = _corners(xr, yr, dxr, dyr, hr)
    bcx, bcy = _corners(xc, yc, dxc, dyc, hc)

    mx = 0.5 * (xr + xc)
    my = 0.5 * (yr + yc)
    acx = [a - mx for a in arx]
    acy = [a - my for a in ary]
    ccx = [b - mx for b in bcx]
    ccy = [b - my for b in bcy]

    acc1 = jnp.zeros_like(mx)
    acc2 = jnp.zeros_like(mx)
    for e in range(4):
        e2 = (e + 1) % 4
        acc1 = _clip_contrib(acx[e], acy[e], acx[e2], acy[e2],
                             tuple(ccx), tuple(ccy), acc1)
        acc2 = _clip_contrib(ccx[e], ccy[e], ccx[e2], ccy[e2],
                             tuple(acx), tuple(acy), acc2)
    inter = 0.5 * (acc1 + acc2)

    ih = jnp.maximum(jnp.minimum(zr + dzr * 0.5, zc + dzc * 0.5)
                     - jnp.maximum(zr - dzr * 0.5, zc - dzc * 0.5), 0.0)
    iv = inter * ih
    va = dxr * dyr * dzr
    vb = dxc * dyc * dzc
    iou = iv / jnp.maximum(va + vb - iv, 1e-6)
    adj_ref[:, :] = (iou > THR).astype(jnp.float32)


def _fuse_kernel(adj_ref, b_ref, bT_ref, sc_ref, sr_ref, t_ref, out_ref):
    f32 = jnp.float32
    lane_n = jax.lax.broadcasted_iota(jnp.int32, (1, NB), 1)
    sub_n = jax.lax.broadcasted_iota(jnp.int32, (NB, 1), 0)

    covered = jnp.zeros((1, NB), f32)
    lead_chunks = []
    lane_c = jax.lax.broadcasted_iota(jnp.int32, (1, CHK), 1)
    for c0 in range(0, NB, CHK):
        def body(g, st):
            covc, leadr = st
            rbase = pl.multiple_of(c0 + g * 8, 8)
            blk = adj_ref[pl.ds(rbase, 8), pl.ds(c0, CHK)]
            for r in range(8):
                oh = lane_c == g * 8 + r
                covk = jnp.sum(jnp.where(oh, covc, 0.0), axis=1,
                               keepdims=True)
                isl = covk == 0.0
                covc = jnp.where(isl & (blk[r:r + 1, :] > 0.0), 1.0, covc)
                leadr = jnp.where(isl & oh, 1.0, leadr)
            return covc, leadr

        covc0 = jax.lax.slice(covered, (0, c0), (1, c0 + CHK))
        _, leadr = jax.lax.fori_loop(
            0, CHK // 8, body, (covc0, jnp.zeros((1, CHK), f32)))
        lead_chunks.append(leadr)
        if c0 + CHK < NB:
            rows = adj_ref[pl.ds(c0, CHK), :]
            hits = jnp.dot(leadr, rows, preferred_element_type=f32)
            covered = jnp.maximum(covered, (hits > 0.0).astype(f32))
    leader_row = jnp.concatenate(lead_chunks, axis=1)

    idn = sub_n == lane_n
    leaderT = jnp.sum(jnp.where(idn, leader_row, 0.0), axis=1, keepdims=True)
    cumT = jnp.sum(jnp.where(lane_n <= sub_n, leader_row, 0.0), axis=1,
                   keepdims=True)
    valT = leaderT * cumT
    adjv = adj_ref[:, :]
    segf_row = (jnp.max(jnp.where(adjv > 0.0, valT, 0.0), axis=0,
                        keepdims=True) - 1.0)
    segT = jnp.sum(jnp.where(idn, segf_row, 0.0), axis=1, keepdims=True)
    same = segT == segf_row
    samef = same.astype(f32)

    s_row = sr_ref[:, :]
    s_col = sc_ref[:, :]

    smax = jnp.max(jnp.where(same, s_row, 0.0), axis=1, keepdims=True)
    sel = same & (s_row == smax)
    lane_f = lane_n.astype(f32)
    amin = jnp.min(jnp.where(sel, lane_f, float(NB)), axis=1, keepdims=True)
    refm = (sel & (lane_f == amin)).astype(f32)
    dirs_row = bT_ref[6:7, :]
    ref_dir = jnp.sum(refm * dirs_row, axis=1, keepdims=True)

    dirsT = b_ref[:, 6:7]
    dd = jnp.abs(dirsT - ref_dir)
    dd = jnp.where(dd > PI_C, 2.0 * PI_C - dd, dd)
    mgt = (dd > PI_C / 2.0).astype(f32)

    x1 = jnp.concatenate([s_col * mgt, s_col * (1.0 - mgt), s_col], axis=1)
    s1 = jnp.dot(samef, x1, preferred_element_type=f32)
    sg, sle, ssum = s1[:, 0:1], s1[:, 1:2], s1[:, 2:3]
    addf = jnp.where(sg <= sle, mgt, 1.0 - mgt)
    dirs2 = _limit_period(dirsT + addf * PI_C)
    snorm = s_col / jnp.where(ssum > 0.0, ssum, 1.0)

    gt = (s_row > s_col) | ((s_row == s_col) & (lane_n < sub_n))
    rank = jnp.sum((same & gt).astype(f32), axis=1, keepdims=True)
    term = jnp.exp((rank + 1.0) * jnp.log(s_col))

    mm = (sub_n.astype(f32) == segf_row).astype(f32)

    a4 = jax.lax.broadcasted_iota(jnp.int32, (4, 1), 0)
    ag = sub_n // NPER_C
    shx = jnp.zeros((NB, 1), f32)
    shy = jnp.zeros((NB, 1), f32)
    for a in range(4):
        sx_a, sy_a = _agent_shift(t_ref, a4 == a)
        shx = jnp.where(ag == a, sx_a, shx)
        shy = jnp.where(ag == a, sy_a, shy)
    bx = b_ref[:, 0:1] + shx
    by = b_ref[:, 1:2] + shy

    x2 = jnp.concatenate(
        [bx * snorm, by * snorm, b_ref[:, 2:3] * snorm, b_ref[:, 3:4] * snorm,
         b_ref[:, 4:5] * snorm, b_ref[:, 5:6] * snorm,
         jnp.sin(dirs2) * snorm, jnp.cos(dirs2) * snorm, term], axis=1)
    s2 = jnp.dot(mm, x2, preferred_element_type=f32)
    theta = jnp.arctan2(s2[:, 6:7], s2[:, 7:8])
    sf = jnp.minimum(s2[:, 8:9], 1.0)
    out_ref[:, :] = jnp.concatenate([s2[:, 0:6], theta, sf], axis=1)


def kernel(det_boxes, det_scores, translations):
    f32 = jnp.float32
    boxes = det_boxes.astype(f32).reshape(NB, 7)
    bT = boxes.T
    s = det_scores.astype(f32).reshape(NB)
    s_col = s.reshape(NB, 1)
    s_row = s.reshape(1, NB)
    t = translations.astype(f32)

    g = NB // TILE
    adj = pl.pallas_call(
        _adj_kernel,
        grid=(g, g),
        in_specs=[
            pl.BlockSpec((TILE, 7), lambda i, j: (i, 0)),
            pl.BlockSpec((7, TILE), lambda i, j: (0, j)),
            pl.BlockSpec((4, 3), lambda i, j: (0, 0)),
        ],
        out_specs=pl.BlockSpec((TILE, TILE), lambda i, j: (i, j)),
        out_shape=jax.ShapeDtypeStruct((NB, NB), f32),
        compiler_params=pltpu.CompilerParams(
            dimension_semantics=("parallel", "arbitrary")),
        interpret=_INTERPRET,
    )(boxes, bT, t)

    out = pl.pallas_call(
        _fuse_kernel,
        out_shape=jax.ShapeDtypeStruct((NB, 8), f32),
        interpret=_INTERPRET,
    )(adj, boxes, bT, s_col, s_row, t)
    return out


# speedup vs baseline: 74.7769x; 1.1994x over previous
import jax
import jax.numpy as jnp
from jax.experimental import pallas as pl
from jax.experimental.pallas import tpu as pltpu

PI_C = 3.141592653
NAG, NPER_C = 4, 256
NB = NAG * NPER_C
THR = 0.1
TILE = 256
CHK = 128

_INTERPRET = False

_SGN = ((0.5, 0.5), (-0.5, 0.5), (-0.5, -0.5), (0.5, -0.5))


def _limit_period(v):
    return v - jnp.floor(v / (2.0 * PI_C) + 0.5) * (2.0 * PI_C)


def _agent_shift(t_ref, sel_mask):
    tx = t_ref[:, 0:1]
    ty = t_ref[:, 1:2]
    a4 = jax.lax.broadcasted_iota(jnp.int32, (4, 1), 0)
    m0 = a4 == 0
    sx = (jnp.sum(jnp.where(sel_mask, tx, 0.0), keepdims=True)
          - jnp.sum(jnp.where(m0, tx, 0.0), keepdims=True))
    sy = (jnp.sum(jnp.where(sel_mask, ty, 0.0), keepdims=True)
          - jnp.sum(jnp.where(m0, ty, 0.0), keepdims=True))
    return sx.reshape(1, 1), sy.reshape(1, 1)


def _clip_contrib(px, py, qx, qy, ux, uy, betas, acc):
    dx, dy = qx - px, qy - py
    t_lo = None
    t_hi = None
    infea = None
    for k in range(4):
        k2 = (k + 1) % 4
        ex = ux[k2] - ux[k]
        ey = uy[k2] - uy[k]
        beta = betas[k]
        alpha = ex * (py - uy[k]) - ey * (px - ux[k])
        r = -alpha / jnp.where(beta == 0.0, 1.0, beta)
        lo_k = jnp.where(beta > 0.0, r, -1e9)
        hi_k = jnp.where(beta < 0.0, r, 1e9)
        bad_k = (beta == 0.0) & (alpha < 0.0)
        if t_lo is None:
            t_lo, t_hi, infea = lo_k, hi_k, bad_k
        else:
            t_lo = jnp.maximum(t_lo, lo_k)
            t_hi = jnp.minimum(t_hi, hi_k)
            infea = infea | bad_k
    t_lo = jnp.maximum(t_lo, 0.0)
    t_hi = jnp.minimum(t_hi, 1.0)
    t_lo = jnp.where(infea, 1e9, t_lo)
    return acc + jnp.where(t_hi > t_lo, (t_hi - t_lo) * (px * dy - py * dx), 0.0)


def _corners(x, y, dx, dy, h):
    c, s = jnp.cos(h), jnp.sin(h)
    cxs, cys = [], []
    for sx, sy in _SGN:
        lx = sx * dx
        ly = sy * dy
        cxs.append(c * lx - s * ly + x)
        cys.append(s * lx + c * ly + y)
    return cxs, cys


def _adj_kernel(br_ref, bcT_ref, t_ref, adj_ref):
    i = pl.program_id(0)
    j = pl.program_id(1)

    @pl.when(i > j)
    def _zero():
        adj_ref[:, :] = jnp.zeros((TILE, TILE), jnp.float32)

    @pl.when(i <= j)
    def _compute():
        _adj_tile(br_ref, bcT_ref, t_ref, adj_ref, i, j)


def _adj_tile(br_ref, bcT_ref, t_ref, adj_ref, i, j):
    a4 = jax.lax.broadcasted_iota(jnp.int32, (4, 1), 0)
    shxr, shyr = _agent_shift(t_ref, a4 == i)
    shxc, shyc = _agent_shift(t_ref, a4 == j)

    br = br_ref[:, :]
    xr = br[:, 0:1] + shxr
    yr = br[:, 1:2] + shyr
    zr, dxr, dyr, dzr, hr = (br[:, 2:3], br[:, 3:4], br[:, 4:5], br[:, 5:6],
                             br[:, 6:7])
    bc = bcT_ref[:, :]
    xc = bc[0:1, :] + shxc
    yc = bc[1:2, :] + shyc
    zc, dxc, dyc, dzc, hc = (bc[2:3, :], bc[3:4, :], bc[4:5, :], bc[5:6, :],
                             bc[6:7, :])

    arx, ary = _corners(xr, yr, dxr, dyr, hr)
    bcx, bcy = _corners(xc, yc, dxc, dyc, hc)

    mx = 0.5 * (xr + xc)
    my = 0.5 * (yr + yc)
    acx = [a - mx for a in arx]
    acy = [a - my for a in ary]
    ccx = [b - mx for b in bcx]
    ccy = [b - my for b in bcy]

    erx = [arx[(e + 1) % 4] - arx[e] for e in range(4)]
    ery = [ary[(e + 1) % 4] - ary[e] for e in range(4)]
    ecx = [bcx[(k + 1) % 4] - bcx[k] for k in range(4)]
    ecy = [bcy[(k + 1) % 4] - bcy[k] for k in range(4)]
    cr = [[ecx[k] * ery[e] - ecy[k] * erx[e] for e in range(4)]
          for k in range(4)]

    acc1 = jnp.zeros_like(mx)
    acc2 = jnp.zeros_like(mx)
    for e in range(4):
        e2 = (e + 1) % 4
        acc1 = _clip_contrib(acx[e], acy[e], acx[e2], acy[e2],
                             tuple(ccx), tuple(ccy),
                             [cr[k][e] for k in range(4)], acc1)
        acc2 = _clip_contrib(ccx[e], ccy[e], ccx[e2], ccy[e2],
                             tuple(acx), tuple(acy),
                             [-cr[e][k] for k in range(4)], acc2)
    inter = 0.5 * (acc1 + acc2)

    ih = jnp.maximum(jnp.minimum(zr + dzr * 0.5, zc + dzc * 0.5)
                     - jnp.maximum(zr - dzr * 0.5, zc - dzc * 0.5), 0.0)
    iv = inter * ih
    va = dxr * dyr * dzr
    vb = dxc * dyc * dzc
    iou = iv / jnp.maximum(va + vb - iv, 1e-6)
    adj_ref[:, :] = (iou > THR).astype(jnp.float32)


def _fuse_kernel(adj_ref, b_ref, bT_ref, sc_ref, sr_ref, t_ref, out_ref,
                 adjs_ref):
    f32 = jnp.float32
    lane_n = jax.lax.broadcasted_iota(jnp.int32, (1, NB), 1)
    sub_n = jax.lax.broadcasted_iota(jnp.int32, (NB, 1), 0)

    adj_u = adj_ref[:, :]
    adjs_ref[:, :] = jnp.maximum(adj_u, adj_u.T)

    covered = jnp.zeros((1, NB), f32)
    lead_chunks = []
    lane_c = jax.lax.broadcasted_iota(jnp.int32, (1, CHK), 1)
    for c0 in range(0, NB, CHK):
        def body(g, st):
            covc, leadr = st
            rbase = pl.multiple_of(c0 + g * 8, 8)
            blk = adjs_ref[pl.ds(rbase, 8), pl.ds(c0, CHK)]
            for r in range(8):
                oh = lane_c == g * 8 + r
                covk = jnp.sum(jnp.where(oh, covc, 0.0), axis=1,
                               keepdims=True)
                isl = covk == 0.0
                covc = jnp.where(isl & (blk[r:r + 1, :] > 0.0), 1.0, covc)
                leadr = jnp.where(isl & oh, 1.0, leadr)
            return covc, leadr

        covc0 = jax.lax.slice(covered, (0, c0), (1, c0 + CHK))
        _, leadr = jax.lax.fori_loop(
            0, CHK // 8, body, (covc0, jnp.zeros((1, CHK), f32)))
        lead_chunks.append(leadr)
        if c0 + CHK < NB:
            rows = adjs_ref[pl.ds(c0, CHK), :]
            hits = jnp.dot(leadr, rows, preferred_element_type=f32)
            covered = jnp.maximum(covered, (hits > 0.0).astype(f32))
    leader_row = jnp.concatenate(lead_chunks, axis=1)

    idn = sub_n == lane_n
    leaderT = jnp.sum(jnp.where(idn, leader_row, 0.0), axis=1, keepdims=True)
    cumT = jnp.sum(jnp.where(lane_n <= sub_n, leader_row, 0.0), axis=1,
                   keepdims=True)
    valT = leaderT * cumT
    adjv = adjs_ref[:, :]
    segf_row = (jnp.max(jnp.where(adjv > 0.0, valT, 0.0), axis=0,
                        keepdims=True) - 1.0)
    segT = jnp.sum(jnp.where(idn, segf_row, 0.0), axis=1, keepdims=True)
    same = segT == segf_row
    samef = same.astype(f32)

    s_row = sr_ref[:, :]
    s_col = sc_ref[:, :]

    smax = jnp.max(jnp.where(same, s_row, 0.0), axis=1, keepdims=True)
    sel = same & (s_row == smax)
    lane_f = lane_n.astype(f32)
    amin = jnp.min(jnp.where(sel, lane_f, float(NB)), axis=1, keepdims=True)
    refm = (sel & (lane_f == amin)).astype(f32)
    dirs_row = bT_ref[6:7, :]
    ref_dir = jnp.sum(refm * dirs_row, axis=1, keepdims=True)

    dirsT = b_ref[:, 6:7]
    dd = jnp.abs(dirsT - ref_dir)
    dd = jnp.where(dd > PI_C, 2.0 * PI_C - dd, dd)
    mgt = (dd > PI_C / 2.0).astype(f32)

    x1 = jnp.concatenate([s_col * mgt, s_col * (1.0 - mgt), s_col], axis=1)
    s1 = jnp.dot(samef, x1, preferred_element_type=f32)
    sg, sle, ssum = s1[:, 0:1], s1[:, 1:2], s1[:, 2:3]
    addf = jnp.where(sg <= sle, mgt, 1.0 - mgt)
    dirs2 = _limit_period(dirsT + addf * PI_C)
    snorm = s_col / jnp.where(ssum > 0.0, ssum, 1.0)

    gt = (s_row > s_col) | ((s_row == s_col) & (lane_n < sub_n))
    rank = jnp.sum((same & gt).astype(f32), axis=1, keepdims=True)
    term = jnp.exp((rank + 1.0) * jnp.log(s_col))

    mm = (sub_n.astype(f32) == segf_row).astype(f32)

    a4 = jax.lax.broadcasted_iota(jnp.int32, (4, 1), 0)
    ag = sub_n // NPER_C
    shx = jnp.zeros((NB, 1), f32)
    shy = jnp.zeros((NB, 1), f32)
    for a in range(4):
        sx_a, sy_a = _agent_shift(t_ref, a4 == a)
        shx = jnp.where(ag == a, sx_a, shx)
        shy = jnp.where(ag == a, sy_a, shy)
    bx = b_ref[:, 0:1] + shx
    by = b_ref[:, 1:2] + shy

    x2 = jnp.concatenate(
        [bx * snorm, by * snorm, b_ref[:, 2:3] * snorm, b_ref[:, 3:4] * snorm,
         b_ref[:, 4:5] * snorm, b_ref[:, 5:6] * snorm,
         jnp.sin(dirs2) * snorm, jnp.cos(dirs2) * snorm, term], axis=1)
    s2 = jnp.dot(mm, x2, preferred_element_type=f32)
    theta = jnp.arctan2(s2[:, 6:7], s2[:, 7:8])
    sf = jnp.minimum(s2[:, 8:9], 1.0)
    out_ref[:, :] = jnp.concatenate([s2[:, 0:6], theta, sf], axis=1)


def kernel(det_boxes, det_scores, translations):
    f32 = jnp.float32
    boxes = det_boxes.astype(f32).reshape(NB, 7)
    bT = boxes.T
    s = det_scores.astype(f32).reshape(NB)
    s_col = s.reshape(NB, 1)
    s_row = s.reshape(1, NB)
    t = translations.astype(f32)

    g = NB // TILE
    adj = pl.pallas_call(
        _adj_kernel,
        grid=(g, g),
        in_specs=[
            pl.BlockSpec((TILE, 7), lambda i, j: (i, 0)),
            pl.BlockSpec((7, TILE), lambda i, j: (0, j)),
            pl.BlockSpec((4, 3), lambda i, j: (0, 0)),
        ],
        out_specs=pl.BlockSpec((TILE, TILE), lambda i, j: (i, j)),
        out_shape=jax.ShapeDtypeStruct((NB, NB), f32),
        compiler_params=pltpu.CompilerParams(
            dimension_semantics=("parallel", "arbitrary")),
        interpret=_INTERPRET,
    )(boxes, bT, t)

    out = pl.pallas_call(
        _fuse_kernel,
        out_shape=jax.ShapeDtypeStruct((NB, 8), f32),
        scratch_shapes=[pltpu.VMEM((NB, NB), f32)],
        interpret=_INTERPRET,
    )(adj, boxes, bT, s_col, s_row, t)
    return out
